# baseline (speedup 1.0000x reference)
import functools

import jax
import jax.numpy as jnp
from jax import lax
from jax.experimental import pallas as pl
from jax.experimental.pallas import tpu as pltpu
from jax.experimental.pallas import tpu_sc as plsc

N_NODES = 10000
NUM_SAMPLE = 32
FEAT_DIM = 128
PACK_DIM = FEAT_DIM // 2

NUM_WORKERS = 32
NODES_PER_WORKER = 320
N_PAD = NUM_WORKERS * NODES_PER_WORKER
CHUNK_NODES = 2
CHUNK_IDX = CHUNK_NODES * NUM_SAMPLE
NUM_CHUNKS = NODES_PER_WORKER // CHUNK_NODES
LANES = 16
W_GROUPS = PACK_DIM // LANES
SCALE = 1.0 / NUM_SAMPLE

ROWS_PER_TILE = N_PAD // 16
OUT_STAGE = 32
OUT_TAIL = N_NODES % OUT_STAGE
CHUNKS_PER_STAGE = OUT_STAGE // CHUNK_NODES
INNER_ITERS = CHUNKS_PER_STAGE // 2
NUM_STAGES = NODES_PER_WORKER // OUT_STAGE

MASK_HI = -65536


def _agg_body(idx_hbm, table_hbm, out_hbm, idx_v, buf0, buf1, outs_a, outs_b,
              tbl_sh, sem0, sem1, osem_a, osem_b):
    sid = lax.axis_index("s")
    wid = sid * 2 + lax.axis_index("c")

    @pl.when(sid < 15)
    def _():
        pltpu.sync_copy(
            table_hbm.at[pl.ds(sid * ROWS_PER_TILE, ROWS_PER_TILE)],
            tbl_sh.at[pl.ds(sid * ROWS_PER_TILE, ROWS_PER_TILE)])

    @pl.when(sid == 15)
    def _():
        pltpu.sync_copy(
            table_hbm.at[pl.ds(15 * ROWS_PER_TILE, N_NODES - 15 * ROWS_PER_TILE)],
            tbl_sh.at[pl.ds(15 * ROWS_PER_TILE, N_NODES - 15 * ROWS_PER_TILE)])

    pltpu.sync_copy(idx_hbm.at[wid], idx_v)
    plsc.subcore_barrier()

    bufs = (buf0, buf1)
    sems = (sem0, sem1)

    def start(c, b):
        pltpu.async_copy(tbl_sh.at[idx_v.at[c]], bufs[b], sems[b])

    def wait(b):
        pltpu.make_async_copy(table_hbm.at[pl.ds(0, CHUNK_IDX)], bufs[b],
                              sems[b]).wait()

    def reduce_chunk(c, pos, b, outs):
        buf = bufs[b]
        sls = [pl.ds(g * LANES, LANES) for g in range(W_GROUPS)]
        for k in range(CHUNK_NODES):
            base = k * NUM_SAMPLE
            acc_e = [None] * W_GROUPS
            acc_o = [None] * W_GROUPS
            for j in range(NUM_SAMPLE):
                for g in range(W_GROUPS):
                    w = lax.bitcast_convert_type(buf[base + j, sls[g]],
                                                 jnp.int32)
                    e = lax.bitcast_convert_type(w << 16, jnp.float32)
                    o = lax.bitcast_convert_type(w & MASK_HI, jnp.float32)
                    if j == 0:
                        acc_e[g] = e
                        acc_o[g] = o
                    else:
                        acc_e[g] = acc_e[g] + e
                        acc_o[g] = acc_o[g] + o
            for g in range(W_GROUPS):
                outs[pos * CHUNK_NODES + k, sls[g]] = acc_e[g] * SCALE
                outs[pos * CHUNK_NODES + k,
                     pl.ds(PACK_DIM + g * LANES, LANES)] = acc_o[g] * SCALE

    start(0, 0)
    start(1, 1)

    def issue_out(row0, outs, osem):
        @pl.when(row0 + OUT_STAGE <= N_NODES)
        def _():
            pltpu.async_copy(outs, out_hbm.at[pl.ds(row0, OUT_STAGE)], osem)

        @pl.when((row0 < N_NODES) & (row0 + OUT_STAGE > N_NODES))
        def _():
            pltpu.async_copy(outs.at[pl.ds(0, OUT_TAIL)],
                             out_hbm.at[pl.ds(row0, OUT_TAIL)], osem)

    def wait_out(row0, outs, osem):
        @pl.when(row0 + OUT_STAGE <= N_NODES)
        def _():
            pltpu.make_async_copy(
                out_hbm.at[pl.ds(0, OUT_STAGE)], outs, osem).wait()

        @pl.when((row0 < N_NODES) & (row0 + OUT_STAGE > N_NODES))
        def _():
            pltpu.make_async_copy(
                out_hbm.at[pl.ds(0, OUT_TAIL)],
                outs.at[pl.ds(0, OUT_TAIL)], osem).wait()

    def half(s, outs, osem, half_idx):
        c_base = s * 2 * CHUNKS_PER_STAGE + half_idx * CHUNKS_PER_STAGE
        row0 = wid * NODES_PER_WORKER + (c_base // CHUNKS_PER_STAGE) * OUT_STAGE

        @pl.when(s > 0)
        def _():
            wait_out(row0 - 2 * OUT_STAGE, outs, osem)

        def inner(j, carry):
            c0 = c_base + 2 * j
            wait(0)
            reduce_chunk(c0, 2 * j, 0, outs)

            @pl.when(c0 + 2 < NUM_CHUNKS)
            def _():
                start(c0 + 2, 0)

            wait(1)
            reduce_chunk(c0 + 1, 2 * j + 1, 1, outs)

            @pl.when(c0 + 3 < NUM_CHUNKS)
            def _():
                start(c0 + 3, 1)

            return carry

        lax.fori_loop(0, INNER_ITERS, inner, 0)
        issue_out(row0, outs, osem)

    def body(s, carry):
        half(s, outs_a, osem_a, 0)
        half(s, outs_b, osem_b, 1)
        return carry

    lax.fori_loop(0, NUM_STAGES // 2, body, 0)

    last = wid * NODES_PER_WORKER + (NUM_STAGES - 2) * OUT_STAGE
    wait_out(last, outs_a, osem_a)
    wait_out(last + OUT_STAGE, outs_b, osem_b)


_mesh = plsc.VectorSubcoreMesh(core_axis_name="c", subcore_axis_name="s")

_agg = functools.partial(
    pl.kernel,
    out_type=jax.ShapeDtypeStruct((N_NODES, FEAT_DIM), jnp.float32),
    mesh=_mesh,
    scratch_types=[
        pltpu.VMEM((NUM_CHUNKS, CHUNK_IDX), jnp.int32),
        pltpu.VMEM((CHUNK_IDX, FEAT_DIM), jnp.float32),
        pltpu.VMEM((CHUNK_IDX, FEAT_DIM), jnp.float32),
        pltpu.VMEM((OUT_STAGE, FEAT_DIM), jnp.float32),
        pltpu.VMEM((OUT_STAGE, FEAT_DIM), jnp.float32),
        pltpu.VMEM_SHARED((N_PAD, FEAT_DIM), jnp.float32),
        pltpu.SemaphoreType.DMA,
        pltpu.SemaphoreType.DMA,
        pltpu.SemaphoreType.DMA,
        pltpu.SemaphoreType.DMA,
    ],
)(_agg_body)


@jax.jit
def kernel(neighbors, table):
    nbr = neighbors.astype(jnp.int32)
    nbr = jnp.pad(nbr, ((0, N_PAD - N_NODES), (0, 0)))
    idx = nbr.reshape(NUM_WORKERS, NUM_CHUNKS, CHUNK_IDX)
    tb = table.astype(jnp.bfloat16)
    pairs = jnp.stack([tb[:, :PACK_DIM], tb[:, PACK_DIM:]], axis=-1)
    packed = lax.bitcast_convert_type(
        lax.bitcast_convert_type(pairs, jnp.int32), jnp.float32)
    wide = jnp.concatenate(
        [packed, jnp.zeros((N_NODES, FEAT_DIM - PACK_DIM), jnp.float32)],
        axis=1)
    return _agg(idx, wide)


# speedup vs baseline: 1.0034x; 1.0034x over previous
import functools

import jax
import jax.numpy as jnp
from jax import lax
from jax.experimental import pallas as pl
from jax.experimental.pallas import tpu as pltpu
from jax.experimental.pallas import tpu_sc as plsc

N_NODES = 10000
NUM_SAMPLE = 32
FEAT_DIM = 128

NUM_WORKERS = 32
NODES_PER_WORKER = 320
N_PAD = NUM_WORKERS * NODES_PER_WORKER
CHUNK_NODES = 2
CHUNK_IDX = CHUNK_NODES * NUM_SAMPLE
NUM_CHUNKS = NODES_PER_WORKER // CHUNK_NODES
LANES = 16
D_BLOCKS = FEAT_DIM // LANES
SCALE = 1.0 / NUM_SAMPLE

ROWS_PER_TILE = N_PAD // 16
OUT_STAGE = 32
OUT_TAIL = N_NODES % OUT_STAGE
CHUNKS_PER_STAGE = OUT_STAGE // CHUNK_NODES
INNER_ITERS = CHUNKS_PER_STAGE // 2
NUM_STAGES = NODES_PER_WORKER // OUT_STAGE


def _agg_body(idx_hbm, table_hbm, out_hbm, idx_v, buf0, buf1, outs_a, outs_b,
              tbl_sh, sem0, sem1, osem_a, osem_b):
    sid = lax.axis_index("s")
    wid = sid * 2 + lax.axis_index("c")

    @pl.when(sid < 15)
    def _():
        pltpu.sync_copy(
            table_hbm.at[pl.ds(sid * ROWS_PER_TILE, ROWS_PER_TILE)],
            tbl_sh.at[pl.ds(sid * ROWS_PER_TILE, ROWS_PER_TILE)])

    @pl.when(sid == 15)
    def _():
        pltpu.sync_copy(
            table_hbm.at[pl.ds(15 * ROWS_PER_TILE, N_NODES - 15 * ROWS_PER_TILE)],
            tbl_sh.at[pl.ds(15 * ROWS_PER_TILE, N_NODES - 15 * ROWS_PER_TILE)])

    pltpu.sync_copy(idx_hbm.at[wid], idx_v)
    plsc.subcore_barrier()

    bufs = (buf0, buf1)
    sems = (sem0, sem1)

    def start(c, b):
        pltpu.async_copy(tbl_sh.at[idx_v.at[c]], bufs[b], sems[b])

    def wait(b):
        pltpu.make_async_copy(table_hbm.at[pl.ds(0, CHUNK_IDX)], bufs[b],
                              sems[b]).wait()

    def reduce_chunk(c, pos, b, outs):
        buf = bufs[b]
        sls = [pl.ds(d * LANES, LANES) for d in range(D_BLOCKS)]
        for k in range(CHUNK_NODES):
            base = k * NUM_SAMPLE
            for g in range(0, D_BLOCKS, 4):
                grp = sls[g:g + 4]
                accs = [buf[base, sl] for sl in grp]
                for j in range(1, NUM_SAMPLE):
                    for d in range(4):
                        accs[d] = accs[d] + buf[base + j, grp[d]]
                for d in range(4):
                    outs[pos * CHUNK_NODES + k, grp[d]] = accs[d] * SCALE

    start(0, 0)
    start(1, 1)

    def issue_out(row0, outs, osem):
        @pl.when(row0 + OUT_STAGE <= N_NODES)
        def _():
            pltpu.async_copy(outs, out_hbm.at[pl.ds(row0, OUT_STAGE)], osem)

        @pl.when((row0 < N_NODES) & (row0 + OUT_STAGE > N_NODES))
        def _():
            pltpu.async_copy(outs.at[pl.ds(0, OUT_TAIL)],
                             out_hbm.at[pl.ds(row0, OUT_TAIL)], osem)

    def wait_out(row0, outs, osem):
        @pl.when(row0 + OUT_STAGE <= N_NODES)
        def _():
            pltpu.make_async_copy(
                table_hbm.at[pl.ds(0, OUT_STAGE)], outs, osem).wait()

        @pl.when((row0 < N_NODES) & (row0 + OUT_STAGE > N_NODES))
        def _():
            pltpu.make_async_copy(
                table_hbm.at[pl.ds(0, OUT_TAIL)],
                outs.at[pl.ds(0, OUT_TAIL)], osem).wait()

    def half(s, outs, osem, half_idx):
        c_base = s * 2 * CHUNKS_PER_STAGE + half_idx * CHUNKS_PER_STAGE
        row0 = wid * NODES_PER_WORKER + (c_base // CHUNKS_PER_STAGE) * OUT_STAGE

        @pl.when(s > 0)
        def _():
            wait_out(row0 - 2 * OUT_STAGE, outs, osem)

        def inner(j, carry):
            c0 = c_base + 2 * j
            wait(0)
            reduce_chunk(c0, 2 * j, 0, outs)

            @pl.when(c0 + 2 < NUM_CHUNKS)
            def _():
                start(c0 + 2, 0)

            wait(1)
            reduce_chunk(c0 + 1, 2 * j + 1, 1, outs)

            @pl.when(c0 + 3 < NUM_CHUNKS)
            def _():
                start(c0 + 3, 1)

            return carry

        lax.fori_loop(0, INNER_ITERS, inner, 0)
        issue_out(row0, outs, osem)

    def body(s, carry):
        half(s, outs_a, osem_a, 0)
        half(s, outs_b, osem_b, 1)
        return carry

    lax.fori_loop(0, NUM_STAGES // 2, body, 0)

    last = wid * NODES_PER_WORKER + (NUM_STAGES - 2) * OUT_STAGE
    wait_out(last, outs_a, osem_a)
    wait_out(last + OUT_STAGE, outs_b, osem_b)


_mesh = plsc.VectorSubcoreMesh(core_axis_name="c", subcore_axis_name="s")

_agg = functools.partial(
    pl.kernel,
    out_type=jax.ShapeDtypeStruct((N_NODES, FEAT_DIM), jnp.float32),
    mesh=_mesh,
    scratch_types=[
        pltpu.VMEM((NUM_CHUNKS, CHUNK_IDX), jnp.int32),
        pltpu.VMEM((CHUNK_IDX, FEAT_DIM), jnp.float32),
        pltpu.VMEM((CHUNK_IDX, FEAT_DIM), jnp.float32),
        pltpu.VMEM((OUT_STAGE, FEAT_DIM), jnp.float32),
        pltpu.VMEM((OUT_STAGE, FEAT_DIM), jnp.float32),
        pltpu.VMEM_SHARED((N_PAD, FEAT_DIM), jnp.float32),
        pltpu.SemaphoreType.DMA,
        pltpu.SemaphoreType.DMA,
        pltpu.SemaphoreType.DMA,
        pltpu.SemaphoreType.DMA,
    ],
)(_agg_body)


@jax.jit
def kernel(neighbors, table):
    nbr = neighbors.astype(jnp.int32)
    nbr = jnp.pad(nbr, ((0, N_PAD - N_NODES), (0, 0)))
    idx = nbr.reshape(NUM_WORKERS, NUM_CHUNKS, CHUNK_IDX)
    return _agg(idx, table)
